# Initial kernel scaffold; baseline (speedup 1.0000x reference)
import jax
import jax.numpy as jnp
from jax.experimental import pallas as pl


def kernel(atomic_numbers, species):
    raise NotImplementedError("write your pallas kernel here")


# speedup vs baseline: 10.2990x; 10.2990x over previous
import functools

import jax
import jax.numpy as jnp
from jax import lax
from jax.experimental import pallas as pl
from jax.experimental.pallas import tpu as pltpu
from jax.experimental.pallas import tpu_sc as plsc

_NUM_WORKERS = 32
_LANES = 16


def _species_lookup_body(n_per_worker, num_species, a_hbm, species_hbm,
                         out_hbm, a_v, out_v, spec_v, lut_v):
  wid = lax.axis_index("s") * 2 + lax.axis_index("c")
  base = wid * n_per_worker

  pltpu.sync_copy(a_hbm.at[pl.ds(base, n_per_worker)], a_v)
  pltpu.sync_copy(species_hbm, spec_v)

  lane = lax.iota(jnp.int32, _LANES)
  plsc.store_scatter(lut_v, [spec_v[...]], lane, mask=lane < num_species)

  def step(g, _):
    idx = a_v[pl.ds(g * _LANES, _LANES)]
    out_v[pl.ds(g * _LANES, _LANES)] = plsc.load_gather(lut_v, [idx])
    return _

  lax.fori_loop(0, n_per_worker // _LANES, step, None, unroll=8)

  pltpu.sync_copy(out_v, out_hbm.at[pl.ds(base, n_per_worker)])


@functools.partial(jax.jit, static_argnames=("n_per_worker", "num_species"))
def _species_lookup(a_pad, species_pad, n_per_worker, num_species):
  mesh = plsc.VectorSubcoreMesh(core_axis_name="c", subcore_axis_name="s")
  body = functools.partial(_species_lookup_body, n_per_worker, num_species)
  return pl.kernel(
      body,
      out_type=jax.ShapeDtypeStruct(a_pad.shape, jnp.int32),
      mesh=mesh,
      scratch_types=[
          pltpu.VMEM((n_per_worker,), jnp.int32),
          pltpu.VMEM((n_per_worker,), jnp.int32),
          pltpu.VMEM((_LANES,), jnp.int32),
          pltpu.VMEM((_LANES,), jnp.int32),
      ],
      compiler_params=pltpu.CompilerParams(needs_layout_passes=False),
  )(a_pad, species_pad)


def kernel(atomic_numbers, species):
  n = atomic_numbers.shape[0]
  num_species = species.shape[0]

  n_per_worker = -(-(-(-n // _NUM_WORKERS)) // _LANES) * _LANES
  n_pad = n_per_worker * _NUM_WORKERS

  a_pad = jnp.concatenate(
      [atomic_numbers.astype(jnp.int32),
       jnp.zeros((n_pad - n,), jnp.int32)])
  species_pad = jnp.concatenate(
      [species.astype(jnp.int32),
       jnp.zeros((_LANES - num_species,), jnp.int32)])

  out = _species_lookup(a_pad, species_pad, n_per_worker, num_species)
  return out[:n]


# speedup vs baseline: 10.8680x; 1.0553x over previous
import functools

import jax
import jax.numpy as jnp
from jax import lax
from jax.experimental import pallas as pl
from jax.experimental.pallas import tpu as pltpu
from jax.experimental.pallas import tpu_sc as plsc

_NUM_WORKERS = 32
_LANES = 16


def _species_lookup_body(chunk, last_chunk, num_species, a_hbm, species_hbm,
                         out_hbm, a_v, out_v, spec_v, lut_v):
  wid = lax.axis_index("s") * 2 + lax.axis_index("c")
  is_last = wid == _NUM_WORKERS - 1
  base = wid * chunk

  @pl.when(jnp.logical_not(is_last))
  def _():
    pltpu.sync_copy(a_hbm.at[pl.ds(base, chunk)], a_v)

  @pl.when(is_last)
  def _():
    pltpu.sync_copy(a_hbm.at[pl.ds(base, last_chunk)],
                    a_v.at[pl.ds(0, last_chunk)])

  spec_v[...] = jnp.zeros((_LANES,), jnp.int32)
  pltpu.sync_copy(species_hbm, spec_v.at[pl.ds(0, num_species)])
  lane = lax.iota(jnp.int32, _LANES)
  plsc.store_scatter(lut_v, [spec_v[...]], lane, mask=lane < num_species)

  def step(g, _):
    idx = a_v[pl.ds(g * _LANES, _LANES)]
    out_v[pl.ds(g * _LANES, _LANES)] = plsc.load_gather(lut_v, [idx])
    return _

  g_last = last_chunk // _LANES
  lax.fori_loop(0, g_last, step, None, unroll=8)

  @pl.when(jnp.logical_not(is_last))
  def _():
    lax.fori_loop(g_last, chunk // _LANES, step, None, unroll=8)
    pltpu.sync_copy(out_v, out_hbm.at[pl.ds(base, chunk)])

  @pl.when(is_last)
  def _():
    pltpu.sync_copy(out_v.at[pl.ds(0, last_chunk)],
                    out_hbm.at[pl.ds(base, last_chunk)])


@functools.partial(jax.jit,
                   static_argnames=("chunk", "last_chunk", "num_species"))
def _species_lookup(a, species, chunk, last_chunk, num_species):
  n = a.shape[0]
  mesh = plsc.VectorSubcoreMesh(core_axis_name="c", subcore_axis_name="s")
  body = functools.partial(_species_lookup_body, chunk, last_chunk,
                           num_species)
  return pl.kernel(
      body,
      out_type=jax.ShapeDtypeStruct((n,), jnp.int32),
      mesh=mesh,
      scratch_types=[
          pltpu.VMEM((chunk,), jnp.int32),
          pltpu.VMEM((chunk,), jnp.int32),
          pltpu.VMEM((_LANES,), jnp.int32),
          pltpu.VMEM((_LANES,), jnp.int32),
      ],
      compiler_params=pltpu.CompilerParams(needs_layout_passes=False),
  )(a, species)


def kernel(atomic_numbers, species):
  n = atomic_numbers.shape[0]
  num_species = species.shape[0]

  chunk = -(-(-(-n // _NUM_WORKERS)) // _LANES) * _LANES
  last_chunk = n - (_NUM_WORKERS - 1) * chunk
  assert last_chunk > 0 and last_chunk % _LANES == 0

  return _species_lookup(atomic_numbers.astype(jnp.int32),
                         species.astype(jnp.int32),
                         chunk, last_chunk, num_species)
